# baseline (speedup 1.0000x reference)
import functools

import jax
import jax.numpy as jnp
from jax.experimental import pallas as pl
from jax.experimental.pallas import tpu as pltpu

H = 128
L = 5
N_TECH = 100000
N_LOC = 50000
N_DEM = 50000
N_FLOW = 100000


def _linear_act_kernel(x_ref, w_ref, b_ref, o_ref, *, act):
    y = jnp.dot(x_ref[...], w_ref[...], preferred_element_type=jnp.float32)
    y = y + b_ref[...]
    if act == "relu":
        y = jnp.maximum(y, 0.0)
    elif act == "tanh":
        y = jnp.tanh(y)
    o_ref[...] = y


def _linear_act(x, W, b, act, block=2048):
    N, din = x.shape
    dout = W.shape[0]
    if din < 8:
        x = jnp.pad(x, ((0, 0), (0, 8 - din)))
        W = jnp.pad(W, ((0, 0), (0, 8 - din)))
        din = 8
    Wt = W.T
    b2 = b.reshape(1, dout)
    grid = (pl.cdiv(N, block),)
    return pl.pallas_call(
        functools.partial(_linear_act_kernel, act=act),
        grid=grid,
        in_specs=[
            pl.BlockSpec((block, din), lambda i: (i, 0)),
            pl.BlockSpec((din, dout), lambda i: (0, 0)),
            pl.BlockSpec((1, dout), lambda i: (0, 0)),
        ],
        out_specs=pl.BlockSpec((block, dout), lambda i: (i, 0)),
        out_shape=jax.ShapeDtypeStruct((N, dout), jnp.float32),
    )(x, Wt, b2)


def _sage_combine_kernel(*refs, K, act):
    s_refs = refs[:K]
    v_refs = refs[K:2 * K]
    x_ref = refs[2 * K]
    w_ref = refs[2 * K + 1]
    b_ref = refs[2 * K + 2]
    o_ref = refs[2 * K + 3]
    acc = jnp.dot(x_ref[...], w_ref[K * H:(K + 1) * H, :],
                  preferred_element_type=jnp.float32)
    for k in range(K):
        s = s_refs[k][...] * v_refs[k][...]
        acc = acc + jnp.dot(s, w_ref[k * H:(k + 1) * H, :],
                            preferred_element_type=jnp.float32)
    acc = acc + b_ref[...]
    if act == "relu":
        acc = jnp.maximum(acc, 0.0)
    else:
        acc = jnp.tanh(acc)
    o_ref[...] = acc


def _sage_combine(ssums, invs, x, Wls, Wr_sum, b_sum, act, block=1024):
    K = len(ssums)
    N = x.shape[0]
    w = jnp.concatenate([Wl.T for Wl in Wls] + [Wr_sum.T], axis=0)
    b2 = b_sum.reshape(1, H)
    grid = (pl.cdiv(N, block),)
    in_specs = (
        [pl.BlockSpec((block, H), lambda i: (i, 0)) for _ in range(K)]
        + [pl.BlockSpec((block, 1), lambda i: (i, 0)) for _ in range(K)]
        + [
            pl.BlockSpec((block, H), lambda i: (i, 0)),
            pl.BlockSpec(((K + 1) * H, H), lambda i: (0, 0)),
            pl.BlockSpec((1, H), lambda i: (0, 0)),
        ]
    )
    return pl.pallas_call(
        functools.partial(_sage_combine_kernel, K=K, act=act),
        grid=grid,
        in_specs=in_specs,
        out_specs=pl.BlockSpec((block, H), lambda i: (i, 0)),
        out_shape=jax.ShapeDtypeStruct((N, H), jnp.float32),
    )(*ssums, *invs, x, w, b2)


def _sort_by_bin(ei, n_dst):
    src = ei[0].astype(jnp.int32)
    dst = ei[1].astype(jnp.int32)
    bink = dst // 16128
    perm = jnp.argsort(bink)
    return jnp.stack([src[perm], dst[perm]])


def _seg_mean_parts(h_src, ei, n_dst):
    src = ei[0]
    dst = ei[1]
    msg = jnp.take(h_src, src, axis=0)
    ssum = jax.ops.segment_sum(msg, dst, num_segments=n_dst)
    return ssum


def _inv_counts(ei, n_dst):
    dst = ei[1]
    cnt = jax.ops.segment_sum(jnp.ones(dst.shape, jnp.float32), dst,
                              num_segments=n_dst)
    return (1.0 / jnp.maximum(cnt, 1.0)).reshape(n_dst, 1)


def kernel(x_technology, x_location, x_demand, x_flow, ei_powers,
           ei_powered_by, ei_feeds, ei_fed_by, ei_connected_to,
           ei_connected_from, enc_W_technology, enc_b_technology,
           enc_W_location, enc_b_location, enc_W_demand, enc_b_demand,
           enc_W_flow, enc_b_flow, conv_Wl, conv_Wr, conv_b, demand_W,
           demand_b, flow_W, flow_b):
    h_tech = _linear_act(x_technology, enc_W_technology, enc_b_technology, "relu")
    h_loc = _linear_act(x_location, enc_W_location, enc_b_location, "relu")
    h_dem = _linear_act(x_demand, enc_W_demand, enc_b_demand, "relu")
    h_flow = _linear_act(x_flow, enc_W_flow, enc_b_flow, "tanh")

    ei_powers = _sort_by_bin(ei_powers, N_LOC)
    ei_fed_by = _sort_by_bin(ei_fed_by, N_LOC)
    ei_connected_to = _sort_by_bin(ei_connected_to, N_LOC)
    ei_powered_by = _sort_by_bin(ei_powered_by, N_TECH)
    ei_feeds = _sort_by_bin(ei_feeds, N_DEM)
    ei_connected_from = _sort_by_bin(ei_connected_from, N_FLOW)

    inv_powers = _inv_counts(ei_powers, N_LOC)
    inv_fed_by = _inv_counts(ei_fed_by, N_LOC)
    inv_conn_to = _inv_counts(ei_connected_to, N_LOC)
    inv_powered_by = _inv_counts(ei_powered_by, N_TECH)
    inv_feeds = _inv_counts(ei_feeds, N_DEM)
    inv_conn_from = _inv_counts(ei_connected_from, N_FLOW)

    for i in range(L):
        s_powers = _seg_mean_parts(h_tech, ei_powers, N_LOC)
        s_fed_by = _seg_mean_parts(h_dem, ei_fed_by, N_LOC)
        s_conn_to = _seg_mean_parts(h_flow, ei_connected_to, N_LOC)
        s_powered_by = _seg_mean_parts(h_loc, ei_powered_by, N_TECH)
        s_feeds = _seg_mean_parts(h_loc, ei_feeds, N_DEM)
        s_conn_from = _seg_mean_parts(h_loc, ei_connected_from, N_FLOW)

        loc_new = _sage_combine(
            [s_powers, s_fed_by, s_conn_to],
            [inv_powers, inv_fed_by, inv_conn_to],
            h_loc,
            [conv_Wl[i, 0], conv_Wl[i, 3], conv_Wl[i, 4]],
            conv_Wr[i, 0] + conv_Wr[i, 3] + conv_Wr[i, 4],
            conv_b[i, 0] + conv_b[i, 3] + conv_b[i, 4],
            "relu")
        tech_new = _sage_combine([s_powered_by], [inv_powered_by], h_tech,
                                 [conv_Wl[i, 1]], conv_Wr[i, 1], conv_b[i, 1],
                                 "relu")
        dem_new = _sage_combine([s_feeds], [inv_feeds], h_dem,
                                [conv_Wl[i, 2]], conv_Wr[i, 2], conv_b[i, 2],
                                "relu")
        flow_new = _sage_combine([s_conn_from], [inv_conn_from], h_flow,
                                 [conv_Wl[i, 5]], conv_Wr[i, 5], conv_b[i, 5],
                                 "tanh")
        h_tech, h_loc, h_dem, h_flow = tech_new, loc_new, dem_new, flow_new

    p_hat = _linear_act(h_tech, demand_W, demand_b, "none")[:, 0]
    flow_pred = _linear_act(h_flow, flow_W, flow_b, "none")[:, 0:1]

    eps = 1e-08
    p_max = x_technology[:, 1] * x_technology[:, 2] * x_technology[:, 3]
    D = jnp.sum(x_demand[:, 0])
    p_bnd = p_max * jax.nn.sigmoid(p_hat)
    S = jnp.sum(p_bnd)
    is_close = jnp.abs(S - D) <= (0.0001 + 1e-05 * jnp.abs(D))
    slack = p_max - p_bnd
    total_slack = jnp.sum(slack) + eps
    shortfall = D - S
    alpha = shortfall / total_slack
    short_out = jnp.where(total_slack < shortfall, p_max, p_bnd + alpha * slack)
    total_bnd = S + eps
    beta = (S - D) / total_bnd
    surp_out = jnp.where(total_bnd < eps, jnp.zeros_like(p_bnd),
                         (1.0 - beta) * p_bnd)
    p_out = jnp.where(is_close, p_bnd, jnp.where(S < D, short_out, surp_out))
    production = p_out[:, None]

    import_cap = x_flow[:, 0:1]
    export_cap = x_flow[:, 1:2]
    flow_out = jnp.where(flow_pred < 0,
                         jax.nn.sigmoid(-flow_pred) * import_cap,
                         jax.nn.sigmoid(flow_pred) * export_cap)
    return (production, flow_out)


# speedup vs baseline: 1.0132x; 1.0132x over previous
import functools

import jax
import jax.numpy as jnp
from jax import lax
from jax.experimental import pallas as pl
from jax.experimental.pallas import tpu as pltpu
from jax.experimental.pallas import tpu_sc as plsc

H = 128
L = 5
N_TECH = 100000
N_LOC = 50000
N_DEM = 50000
N_FLOW = 100000

_C = 128
_RANGE = 13312
_ACC = 13440
_CH = _RANGE // 16
_ZB = 48


def _round_up(x, m):
    return (x + m - 1) // m * m


def _mxu_dot(a, b):
    return jnp.dot(a, b, preferred_element_type=jnp.float32)


def _linear_act_kernel(x_ref, w_ref, b_ref, o_ref, *, act, prec):
    y = jnp.dot(x_ref[...], w_ref[...], preferred_element_type=jnp.float32,
                precision=prec)
    y = y + b_ref[...]
    if act == "relu":
        y = jnp.maximum(y, 0.0)
    elif act == "tanh":
        y = jnp.tanh(y)
    o_ref[...] = y


def _linear_act(x, W, b, act, block=2048, prec=None):
    N, din = x.shape
    dout = W.shape[0]
    if din < 8:
        x = jnp.pad(x, ((0, 0), (0, 8 - din)))
        W = jnp.pad(W, ((0, 0), (0, 8 - din)))
        din = 8
    Wt = W.T
    b2 = b.reshape(1, dout)
    grid = (pl.cdiv(N, block),)
    return pl.pallas_call(
        functools.partial(_linear_act_kernel, act=act, prec=prec),
        grid=grid,
        in_specs=[
            pl.BlockSpec((block, din), lambda i: (i, 0)),
            pl.BlockSpec((din, dout), lambda i: (0, 0)),
            pl.BlockSpec((1, dout), lambda i: (0, 0)),
        ],
        out_specs=pl.BlockSpec((block, dout), lambda i: (i, 0)),
        out_shape=jax.ShapeDtypeStruct((N, dout), jnp.float32),
    )(x, Wt, b2)


def _sage_combine_kernel(*refs, K, act):
    s_refs = refs[:K]
    v_refs = refs[K:2 * K]
    x_ref = refs[2 * K]
    w_ref = refs[2 * K + 1]
    b_ref = refs[2 * K + 2]
    o_ref = refs[2 * K + 3]
    acc = None
    x = x_ref[...]
    for k in range(K):
        agg = s_refs[k][...] / v_refs[k][...]
        term = _mxu_dot(agg, w_ref[2 * k * H:(2 * k + 1) * H, :])
        term = term + _mxu_dot(x, w_ref[(2 * k + 1) * H:(2 * k + 2) * H, :])
        term = term + b_ref[k, :][None, :]
        acc = term if acc is None else acc + term
    if act == "relu":
        acc = jnp.maximum(acc, 0.0)
    o_ref[...] = acc


def _sage_combine(ssums, cnts, x, Wls, Wrs, bs, act, block=1024):
    K = len(ssums)
    N = x.shape[0]
    wlist = []
    for Wl, Wr in zip(Wls, Wrs):
        wlist.append(Wl.T)
        wlist.append(Wr.T)
    w = jnp.concatenate(wlist, axis=0)
    b2 = jnp.stack(bs, axis=0)
    grid = (pl.cdiv(N, block),)
    in_specs = (
        [pl.BlockSpec((block, H), lambda i: (i, 0)) for _ in range(K)]
        + [pl.BlockSpec((block, 1), lambda i: (i, 0)) for _ in range(K)]
        + [
            pl.BlockSpec((block, H), lambda i: (i, 0)),
            pl.BlockSpec((2 * K * H, H), lambda i: (0, 0)),
            pl.BlockSpec((K, H), lambda i: (0, 0)),
        ]
    )
    return pl.pallas_call(
        functools.partial(_sage_combine_kernel, K=K, act=act),
        grid=grid,
        in_specs=in_specs,
        out_specs=pl.BlockSpec((block, H), lambda i: (i, 0)),
        out_shape=jax.ShapeDtypeStruct((N, H), jnp.float32),
    )(*ssums, *cnts, x, w, b2)


_TYPES = (
    (0, 100000, N_LOC),
    (2, 50000, N_LOC),
    (3, 200000, N_LOC),
    (1, 100000, N_TECH),
    (1, 50000, N_DEM),
    (1, 200000, N_FLOW),
)


def _type_R(n_dst):
    return pl.cdiv(n_dst, _RANGE)


def _prep_type(ei, n_dst):
    E = ei.shape[1]
    R = _type_R(n_dst)
    src = ei[0].astype(jnp.int32)
    dst = ei[1].astype(jnp.int32)
    binr = dst // _RANGE
    dloc = dst - binr * _RANGE
    order = jnp.argsort(binr, stable=True)
    src_s = src[order]
    dloc_s = dloc[order]
    bins_s = binr[order]
    bounds = jnp.searchsorted(
        bins_s, jnp.arange(R + 1, dtype=jnp.int32)).astype(jnp.int32)
    cnt = bounds[1:] - bounds[:-1]
    pcnt = (cnt + _C - 1) // _C * _C
    start = jnp.concatenate(
        [jnp.zeros((1,), jnp.int32), jnp.cumsum(pcnt).astype(jnp.int32)])
    pos = start[bins_s] + jnp.arange(E, dtype=jnp.int32) - bounds[bins_s]
    E_cap = _round_up(E, _C) + R * _C
    sidx = jnp.zeros((E_cap,), jnp.int32).at[pos].set(src_s)
    didx = (_RANGE + (jnp.arange(E_cap, dtype=jnp.int32) % 64)).at[pos].set(dloc_s)
    bb = jnp.pad(start // _C, (0, 32 - (R + 1)), mode="edge")
    brows = bb[jnp.arange(8)[:, None] + jnp.arange(16)[None, :]]
    return (sidx.reshape(E_cap // _C, _C), didx.reshape(E_cap // _C, _C),
            brows.reshape(128))


def _make_layer_segsum():
    mesh = plsc.VectorSubcoreMesh(core_axis_name="c", subcore_axis_name="s")
    out_types = tuple(
        jax.ShapeDtypeStruct((_type_R(n_dst) * _RANGE, H), jnp.float32)
        for (_, _, n_dst) in _TYPES)

    @functools.partial(
        pl.kernel,
        out_type=out_types,
        mesh=mesh,
        scratch_types=[
            pltpu.VMEM((1, _C), jnp.int32),
            pltpu.VMEM((1, _C), jnp.int32),
            pltpu.VMEM((_C, H), jnp.float32),
            pltpu.VMEM((_ZB, H), jnp.float32),
            pltpu.VMEM((128,), jnp.int32),
            pltpu.VMEM_SHARED((_ACC, H), jnp.float32),
        ],
    )
    def segsum(h_tech, h_loc, h_dem, h_flow,
               s0, d0, b0, s1, d1, b1, s2, d2, b2,
               s3, d3, b3, s4, d4, b4, s5, d5, b5,
               zeros_hbm,
               o0, o1, o2, o3, o4, o5,
               sidx_v, didx_v, gbuf, zbuf, bnds, acc):
        c = lax.axis_index("c")
        t = lax.axis_index("s")
        tables = (h_tech, h_loc, h_dem, h_flow)
        sds = ((s0, d0, b0), (s1, d1, b1), (s2, d2, b2),
               (s3, d3, b3), (s4, d4, b4), (s5, d5, b5))
        outs = (o0, o1, o2, o3, o4, o5)
        pltpu.sync_copy(zeros_hbm, zbuf)

        for ti, (hi, E, n_dst) in enumerate(_TYPES):
            h_hbm = tables[hi]
            sidx_hbm, didx_hbm, bounds_hbm = sds[ti]
            out_hbm = outs[ti]
            R = _type_R(n_dst)
            pltpu.sync_copy(bounds_hbm, bnds)

            for ri in range(pl.cdiv(R, 2)):
                r = 2 * ri + c

                @pl.when(r < R)
                def _(r=r, h_hbm=h_hbm, sidx_hbm=sidx_hbm,
                      didx_hbm=didx_hbm, out_hbm=out_hbm):
                    bv = bnds[pl.ds(r * 16, 16)]
                    B0 = bv[0]
                    B1 = bv[1]
                    for z in range(_CH // _ZB):
                        pltpu.sync_copy(
                            zbuf, acc.at[pl.ds(t * _CH + z * _ZB, _ZB)])
                    rem = _CH % _ZB
                    if rem:
                        pltpu.sync_copy(
                            zbuf.at[pl.ds(0, rem)],
                            acc.at[pl.ds(t * _CH + (_CH // _ZB) * _ZB, rem)])
                    plsc.subcore_barrier()

                    nb = B1 - B0
                    nb_my = (nb - t + 15) // 16

                    def batch(k, carry):
                        kb = B0 + t + k * 16
                        pltpu.sync_copy(sidx_hbm.at[pl.ds(kb, 1)], sidx_v)
                        pltpu.sync_copy(didx_hbm.at[pl.ds(kb, 1)], didx_v)
                        pltpu.sync_copy(h_hbm.at[sidx_v.at[0]], gbuf)
                        pltpu.sync_copy(gbuf, acc.at[didx_v.at[0]], add=True)
                        return carry

                    lax.fori_loop(0, nb_my, batch, 0)
                    plsc.subcore_barrier()
                    pltpu.sync_copy(
                        acc.at[pl.ds(t * _CH, _CH)],
                        out_hbm.at[pl.ds(r * _RANGE + t * _CH, _CH)])

            plsc.subcore_barrier()

    return segsum


def _max_counts(ei, n_dst):
    dst = ei[1]
    cnt = jax.ops.segment_sum(jnp.ones(dst.shape, jnp.float32), dst,
                              num_segments=n_dst)
    return jnp.maximum(cnt, 1.0).reshape(n_dst, 1)


def kernel(x_technology, x_location, x_demand, x_flow, ei_powers,
           ei_powered_by, ei_feeds, ei_fed_by, ei_connected_to,
           ei_connected_from, enc_W_technology, enc_b_technology,
           enc_W_location, enc_b_location, enc_W_demand, enc_b_demand,
           enc_W_flow, enc_b_flow, conv_Wl, conv_Wr, conv_b, demand_W,
           demand_b, flow_W, flow_b):
    h_tech = _linear_act(x_technology, enc_W_technology, enc_b_technology, "relu")
    h_loc = _linear_act(x_location, enc_W_location, enc_b_location, "relu")
    h_dem = _linear_act(x_demand, enc_W_demand, enc_b_demand, "relu")
    h_flow = jnp.tanh(_linear_act(x_flow, enc_W_flow, enc_b_flow, "none"))

    eis = (ei_powers, ei_fed_by, ei_connected_to,
           ei_powered_by, ei_feeds, ei_connected_from)
    preps = [_prep_type(ei, n_dst)
             for ei, (_, _, n_dst) in zip(eis, _TYPES)]
    flat = []
    for p in preps:
        flat.extend(p)
    zeros = jnp.zeros((_ZB, H), jnp.float32)

    cnt_powers = _max_counts(ei_powers, N_LOC)
    cnt_fed_by = _max_counts(ei_fed_by, N_LOC)
    cnt_conn_to = _max_counts(ei_connected_to, N_LOC)
    cnt_powered_by = _max_counts(ei_powered_by, N_TECH)
    cnt_feeds = _max_counts(ei_feeds, N_DEM)
    cnt_conn_from = _max_counts(ei_connected_from, N_FLOW)

    segsum = _make_layer_segsum()

    for i in range(L):
        def _xs(h, ei, n_dst):
            return jax.ops.segment_sum(
                jnp.take(h, ei[0], axis=0), ei[1], num_segments=n_dst)
        s_powers = _xs(h_tech, ei_powers, N_LOC)
        s_fed_by = _xs(h_dem, ei_fed_by, N_LOC)
        s_conn_to = _xs(h_flow, ei_connected_to, N_LOC)
        s_powered_by = _xs(h_loc, ei_powered_by, N_TECH)
        s_feeds = _xs(h_loc, ei_feeds, N_DEM)
        s_conn_from = _xs(h_loc, ei_connected_from, N_FLOW)

        loc_new = _sage_combine(
            [s_powers, s_fed_by, s_conn_to],
            [cnt_powers, cnt_fed_by, cnt_conn_to],
            h_loc,
            [conv_Wl[i, 0], conv_Wl[i, 3], conv_Wl[i, 4]],
            [conv_Wr[i, 0], conv_Wr[i, 3], conv_Wr[i, 4]],
            [conv_b[i, 0], conv_b[i, 3], conv_b[i, 4]],
            "relu")
        tech_new = _sage_combine([s_powered_by], [cnt_powered_by], h_tech,
                                 [conv_Wl[i, 1]], [conv_Wr[i, 1]],
                                 [conv_b[i, 1]], "relu")
        dem_new = _sage_combine([s_feeds], [cnt_feeds], h_dem,
                                [conv_Wl[i, 2]], [conv_Wr[i, 2]],
                                [conv_b[i, 2]], "relu")
        flow_new = jnp.tanh(
            _sage_combine([s_conn_from], [cnt_conn_from], h_flow,
                          [conv_Wl[i, 5]], [conv_Wr[i, 5]],
                          [conv_b[i, 5]], "none"))
        h_tech, h_loc, h_dem, h_flow = tech_new, loc_new, dem_new, flow_new

    p_hat = _linear_act(h_tech, demand_W, demand_b, "none",
                        prec=jax.lax.Precision.HIGHEST)[:, 0]
    flow_pred = _linear_act(h_flow, flow_W, flow_b, "none",
                            prec=jax.lax.Precision.HIGHEST)[:, 0:1]

    eps = 1e-08
    p_max = x_technology[:, 1] * x_technology[:, 2] * x_technology[:, 3]
    D = jnp.sum(x_demand[:, 0])
    p_bnd = p_max * jax.nn.sigmoid(p_hat)
    S = jnp.sum(p_bnd)
    is_close = jnp.abs(S - D) <= (0.0001 + 1e-05 * jnp.abs(D))
    slack = p_max - p_bnd
    total_slack = jnp.sum(slack) + eps
    shortfall = D - S
    alpha = shortfall / total_slack
    short_out = jnp.where(total_slack < shortfall, p_max, p_bnd + alpha * slack)
    total_bnd = S + eps
    beta = (S - D) / total_bnd
    surp_out = jnp.where(total_bnd < eps, jnp.zeros_like(p_bnd),
                         (1.0 - beta) * p_bnd)
    p_out = jnp.where(is_close, p_bnd, jnp.where(S < D, short_out, surp_out))
    production = p_out[:, None]

    import_cap = x_flow[:, 0:1]
    export_cap = x_flow[:, 1:2]
    flow_out = jnp.where(flow_pred < 0,
                         jax.nn.sigmoid(-flow_pred) * import_cap,
                         jax.nn.sigmoid(flow_pred) * export_cap)
    return (production, flow_out)
